# baseline (speedup 1.0000x reference)
import functools

import jax
import jax.numpy as jnp
from jax import lax
from jax.experimental import pallas as pl
from jax.experimental.pallas import tpu as pltpu
from jax.experimental.pallas import tpu_sc as plsc

_NC = 2
_NS = 16
_NW = _NC * _NS


@functools.lru_cache(maxsize=None)
def _make_gather(B, D, chunk, nbuf):
    b_per_w = B // _NW
    sizes = [chunk] * (b_per_w // chunk - 1)
    rem = b_per_w - sum(sizes)
    for s in (chunk // 2, chunk // 4, chunk // 8):
        s -= s % 8
        sizes.append(s)
        rem -= s
    sizes.append(rem)
    assert rem > 0 and rem % 8 == 0 and all(s % 8 == 0 for s in sizes)
    offs = [0]
    for s in sizes:
        offs.append(offs[-1] + s)
    n_chunks = len(sizes)
    mesh = plsc.VectorSubcoreMesh(core_axis_name="c", subcore_axis_name="s")

    @functools.partial(
        pl.kernel,
        out_type=jax.ShapeDtypeStruct((B, D), jnp.float32),
        mesh=mesh,
        scratch_types=(
            [pltpu.VMEM((chunk,), jnp.int32)] * nbuf
            + [pltpu.VMEM((chunk, D), jnp.float32)] * nbuf
            + [pltpu.SemaphoreType.DMA] * (3 * nbuf)
        ),
        compiler_params=pltpu.CompilerParams(use_tc_tiling_on_sc=False),
    )
    def gather_kernel(x_hbm, w_hbm, out_hbm, *scratch):
        idxb = scratch[:nbuf]
        rows = scratch[nbuf:2 * nbuf]
        isem = scratch[2 * nbuf:3 * nbuf]
        gsem = scratch[3 * nbuf:4 * nbuf]
        wsem = scratch[4 * nbuf:]
        sub = lax.axis_index("s")
        wid = sub * _NC + lax.axis_index("c")
        base0 = wid * b_per_w

        def idx_copy(j, b):
            return pltpu.async_copy(
                x_hbm.at[pl.ds(base0 + offs[j], sizes[j])],
                idxb[b].at[pl.ds(0, sizes[j])], isem[b])

        def gather_desc(j, b):
            return pltpu.make_async_copy(
                w_hbm.at[idxb[b].at[pl.ds(0, sizes[j])]],
                rows[b].at[pl.ds(0, sizes[j])], gsem[b])

        def writeback(j, b):
            return pltpu.async_copy(
                rows[b].at[pl.ds(0, sizes[j])],
                out_hbm.at[pl.ds(base0 + offs[j], sizes[j])], wsem[b])

        for j in range(min(nbuf, n_chunks)):
            idx_copy(j, j)
        for j in range(min(nbuf - 1, n_chunks)):
            pltpu.make_async_copy(
                x_hbm.at[pl.ds(base0 + offs[j], sizes[j])],
                idxb[j].at[pl.ds(0, sizes[j])], isem[j]).wait()
            gather_desc(j, j).start()

        pending_w = [None] * nbuf
        for i in range(n_chunks):
            b = i % nbuf
            j = i + nbuf - 1
            if j < n_chunks:
                bj = j % nbuf
                if pending_w[bj] is not None:
                    pending_w[bj].wait()
                    pending_w[bj] = None
                pltpu.make_async_copy(
                    x_hbm.at[pl.ds(base0 + offs[j], sizes[j])],
                    idxb[bj].at[pl.ds(0, sizes[j])], isem[bj]).wait()
                gather_desc(j, bj).start()
            gather_desc(i, b).wait()
            pending_w[b] = writeback(i, b)
            if i + nbuf < n_chunks:
                idx_copy(i + nbuf, b)
        for b in range(nbuf):
            if pending_w[b] is not None:
                pending_w[b].wait()

    return gather_kernel


def kernel(x, W):
    B0, H = x.shape
    V, D = W.shape
    B = B0 * H
    flat_x = x.reshape((B,)).astype(jnp.int32)
    out = _make_gather(B, D, 800, 4)(flat_x, W)
    return out.reshape((B0, H, D))
